# baseline (speedup 1.0000x reference)
import functools

import jax
import jax.numpy as jnp
from jax import lax
from jax.experimental import pallas as pl
from jax.experimental.pallas import tpu as pltpu
from jax.experimental.pallas import tpu_sc as plsc

_SEG = 2048
_HW = 196
_SND_CHUNK = 128


def _make_sc_kernel(N, C, n_seg):
    info = plsc.get_sparse_core_info()
    nw = info.num_cores * info.num_subcores
    halves = nw // n_seg
    rows_w = N // nw
    nk = rows_w // _SND_CHUNK
    ng = C // 16
    mesh = plsc.VectorSubcoreMesh(core_axis_name="c", subcore_axis_name="s")

    @functools.partial(
        pl.kernel,
        out_type=jax.ShapeDtypeStruct((n_seg, halves, C), jnp.float32),
        mesh=mesh,
        scratch_types=[
            pltpu.VMEM((2, _SND_CHUNK, C), jnp.float32),
            pltpu.VMEM((C,), jnp.float32),
            pltpu.SemaphoreType.DMA,
            pltpu.SemaphoreType.DMA,
        ],
    )
    def seg_sums(z_hbm, out_hbm, buf, row_v, sem0, sem1):
        wid = lax.axis_index("s") * info.num_cores + lax.axis_index("c")
        base = wid * rows_w
        sems = (sem0, sem1)

        def copy(k):
            return pltpu.make_async_copy(
                z_hbm.at[pl.ds(base + k * _SND_CHUNK, _SND_CHUNK), :],
                buf.at[k % 2], sems[k % 2])

        copy(0).start()
        accs = tuple(jnp.zeros((16,), jnp.float32) for _ in range(ng))
        for k in range(nk):
            if k + 1 < nk:
                copy(k + 1).start()
            copy(k).wait()
            slot = buf.at[k % 2]

            def body(i, a, slot=slot):
                r = i * 4
                for u in range(4):
                    a = tuple(
                        a[c] + slot[r + u, c * 16:(c + 1) * 16]
                        for c in range(ng))
                return a

            accs = lax.fori_loop(0, _SND_CHUNK // 4, body, accs)
        for c in range(ng):
            row_v[c * 16:(c + 1) * 16] = accs[c]
        pltpu.sync_copy(row_v, out_hbm.at[wid // halves, wid % halves])

    return seg_sums


def _fused_body(inv_ref, x_ref, w_ref, snd_ref, mimg_ref, msnd_ref):
    gw = _HW * 32
    parts = [
        jnp.dot(x_ref[:, g * gw:(g + 1) * gw], w_ref[...],
                preferred_element_type=jnp.float32)
        for g in range(x_ref.shape[1] // gw)
    ]
    m = jnp.concatenate(parts, axis=1) * (1.0 / _HW)
    mimg_ref[...] = jnp.broadcast_to(m[None, :, :], mimg_ref.shape)
    rows = jnp.sum(snd_ref[...], axis=1, keepdims=True) * inv_ref[0]
    msnd_ref[...] = jnp.broadcast_to(rows, msnd_ref.shape)


def kernel(Z_img, Z_snd, snd_splits):
    B, C, H, W = Z_img.shape
    N = Z_snd.shape[0]
    n_seg = N // _SEG

    snd_part = _make_sc_kernel(N, C, n_seg)(Z_snd)

    HW = H * W
    Z_img_flat = Z_img.reshape(B, C * HW)
    gw = HW * 32
    w_sum = jax.nn.one_hot(jnp.arange(gw) // HW, 32, dtype=jnp.float32)
    inv = (1.0 / jnp.asarray(snd_splits).astype(jnp.float32)).reshape(1)
    M_img, M_snd = pl.pallas_call(
        _fused_body,
        grid=(B // 8,),
        in_specs=[
            pl.BlockSpec(memory_space=pltpu.SMEM),
            pl.BlockSpec((8, C * HW), lambda i: (i, 0)),
            pl.BlockSpec((gw, 32), lambda i: (0, 0)),
            pl.BlockSpec((n_seg, 2, C), lambda i: (0, 0, 0)),
        ],
        out_specs=[
            pl.BlockSpec((n_seg, 8, C), lambda i: (0, i, 0)),
            pl.BlockSpec((n_seg, 8, C), lambda i: (0, i, 0)),
        ],
        out_shape=[
            jax.ShapeDtypeStruct((n_seg, B, C), jnp.float32),
            jax.ShapeDtypeStruct((n_seg, B, C), jnp.float32),
        ],
    )(inv, Z_img_flat, w_sum, snd_part)
    return (M_img, M_snd)


# speedup vs baseline: 1.9016x; 1.9016x over previous
import functools

import jax
import jax.numpy as jnp
from jax import lax
from jax.experimental import pallas as pl
from jax.experimental.pallas import tpu as pltpu
from jax.experimental.pallas import tpu_sc as plsc

_SEG = 2048
_HW = 196
_SND_CHUNK = 128


def _make_sc_kernel(N, C, n_seg):
    info = plsc.get_sparse_core_info()
    nw = info.num_cores * info.num_subcores
    halves = nw // n_seg
    rows_w = N // nw
    nk = rows_w // _SND_CHUNK
    ng = C // 16
    mesh = plsc.VectorSubcoreMesh(core_axis_name="c", subcore_axis_name="s")

    @functools.partial(
        pl.kernel,
        out_type=jax.ShapeDtypeStruct((n_seg, halves, C), jnp.float32),
        mesh=mesh,
        scratch_types=[
            pltpu.VMEM((2, _SND_CHUNK, C), jnp.float32),
            pltpu.VMEM((C,), jnp.float32),
            pltpu.SemaphoreType.DMA,
            pltpu.SemaphoreType.DMA,
        ],
    )
    def seg_sums(z_hbm, out_hbm, buf, row_v, sem0, sem1):
        wid = lax.axis_index("s") * info.num_cores + lax.axis_index("c")
        base = wid * rows_w
        sems = (sem0, sem1)

        def copy(k):
            return pltpu.make_async_copy(
                z_hbm.at[pl.ds(base + k * _SND_CHUNK, _SND_CHUNK), :],
                buf.at[k % 2], sems[k % 2])

        copy(0).start()
        accs = tuple(jnp.zeros((16,), jnp.float32) for _ in range(ng))
        for k in range(nk):
            if k + 1 < nk:
                copy(k + 1).start()
            copy(k).wait()
            slot = buf.at[k % 2]

            def body(i, a, slot=slot):
                r = i * 4
                for u in range(4):
                    a = tuple(
                        a[c] + slot[r + u, c * 16:(c + 1) * 16]
                        for c in range(ng))
                return a

            accs = lax.fori_loop(0, _SND_CHUNK // 4, body, accs)
        for c in range(ng):
            row_v[c * 16:(c + 1) * 16] = accs[c]
        pltpu.sync_copy(row_v, out_hbm.at[wid // halves, wid % halves])

    return seg_sums


def _fused_body(inv_ref, x_ref, snd_ref, mimg_ref, msnd_ref):
    m = jnp.sum(x_ref[...], axis=2) * (1.0 / _HW)
    mimg_ref[...] = jnp.broadcast_to(m[None, :, :], mimg_ref.shape)
    rows = jnp.sum(snd_ref[...], axis=1, keepdims=True) * inv_ref[0]
    msnd_ref[...] = jnp.broadcast_to(rows, msnd_ref.shape)


def kernel(Z_img, Z_snd, snd_splits):
    B, C, H, W = Z_img.shape
    N = Z_snd.shape[0]
    n_seg = N // _SEG

    snd_part = _make_sc_kernel(N, C, n_seg)(Z_snd)

    Z_img_flat = Z_img.reshape(B, C, H * W)
    inv = (1.0 / jnp.asarray(snd_splits).astype(jnp.float32)).reshape(1)
    M_img, M_snd = pl.pallas_call(
        _fused_body,
        grid=(B // 8,),
        in_specs=[
            pl.BlockSpec(memory_space=pltpu.SMEM),
            pl.BlockSpec((8, C, H * W), lambda i: (i, 0, 0)),
            pl.BlockSpec((n_seg, 2, C), lambda i: (0, 0, 0)),
        ],
        out_specs=[
            pl.BlockSpec((n_seg, 8, C), lambda i: (0, i, 0)),
            pl.BlockSpec((n_seg, 8, C), lambda i: (0, i, 0)),
        ],
        out_shape=[
            jax.ShapeDtypeStruct((n_seg, B, C), jnp.float32),
            jax.ShapeDtypeStruct((n_seg, B, C), jnp.float32),
        ],
    )(inv, Z_img_flat, snd_part)
    return (M_img, M_snd)


# speedup vs baseline: 1.9572x; 1.0292x over previous
import functools

import jax
import jax.numpy as jnp
from jax import lax
from jax.experimental import pallas as pl
from jax.experimental.pallas import tpu as pltpu
from jax.experimental.pallas import tpu_sc as plsc

_SEG = 2048
_HW = 196
_SND_CHUNK = 128


def _make_sc_kernel(N, C, n_seg):
    info = plsc.get_sparse_core_info()
    nw = info.num_cores * info.num_subcores
    halves = nw // n_seg
    rows_w = N // nw
    nk = rows_w // _SND_CHUNK
    ng = C // 16
    mesh = plsc.VectorSubcoreMesh(core_axis_name="c", subcore_axis_name="s")

    @functools.partial(
        pl.kernel,
        out_type=jax.ShapeDtypeStruct((n_seg, halves, C), jnp.float32),
        mesh=mesh,
        scratch_types=[
            pltpu.VMEM((2, _SND_CHUNK, C), jnp.float32),
            pltpu.VMEM((C,), jnp.float32),
            pltpu.SemaphoreType.DMA,
            pltpu.SemaphoreType.DMA,
        ],
    )
    def seg_sums(z_hbm, out_hbm, buf, row_v, sem0, sem1):
        wid = lax.axis_index("s") * info.num_cores + lax.axis_index("c")
        base = wid * rows_w
        sems = (sem0, sem1)

        def copy(k):
            return pltpu.make_async_copy(
                z_hbm.at[pl.ds(base + k * _SND_CHUNK, _SND_CHUNK), :],
                buf.at[k % 2], sems[k % 2])

        copy(0).start()
        accs = tuple(jnp.zeros((16,), jnp.float32) for _ in range(ng))
        for k in range(nk):
            if k + 1 < nk:
                copy(k + 1).start()
            copy(k).wait()
            slot = buf.at[k % 2]

            def body(i, a, slot=slot):
                r = i * 4
                for u in range(4):
                    a = tuple(
                        a[c] + slot[r + u, c * 16:(c + 1) * 16]
                        for c in range(ng))
                return a

            accs = lax.fori_loop(0, _SND_CHUNK // 4, body, accs)
        for c in range(ng):
            row_v[c * 16:(c + 1) * 16] = accs[c]
        pltpu.sync_copy(row_v, out_hbm.at[wid // halves, wid % halves])

    return seg_sums


def _img_body(x_ref, mimg_ref):
    m = jnp.sum(x_ref[...], axis=2) * (1.0 / _HW)
    mimg_ref[...] = jnp.broadcast_to(m[None, :, :], mimg_ref.shape)


def _msnd_body(inv_ref, snd_ref, msnd_ref):
    rows = jnp.sum(snd_ref[...], axis=1, keepdims=True) * inv_ref[0]
    msnd_ref[...] = jnp.broadcast_to(rows, msnd_ref.shape)


def kernel(Z_img, Z_snd, snd_splits):
    B, C, H, W = Z_img.shape
    N = Z_snd.shape[0]
    n_seg = N // _SEG

    snd_part = _make_sc_kernel(N, C, n_seg)(Z_snd)

    Z_img_flat = Z_img.reshape(B, C, H * W)
    M_img = pl.pallas_call(
        _img_body,
        grid=(B // 8,),
        in_specs=[pl.BlockSpec((8, C, H * W), lambda i: (i, 0, 0))],
        out_specs=pl.BlockSpec((n_seg, 8, C), lambda i: (0, i, 0)),
        out_shape=jax.ShapeDtypeStruct((n_seg, B, C), jnp.float32),
    )(Z_img_flat)

    inv = (1.0 / jnp.asarray(snd_splits).astype(jnp.float32)).reshape(1)
    M_snd = pl.pallas_call(
        _msnd_body,
        grid=(B // 8,),
        in_specs=[
            pl.BlockSpec(memory_space=pltpu.SMEM),
            pl.BlockSpec((n_seg, 2, C), lambda i: (0, 0, 0)),
        ],
        out_specs=pl.BlockSpec((n_seg, 8, C), lambda i: (0, i, 0)),
        out_shape=jax.ShapeDtypeStruct((n_seg, B, C), jnp.float32),
    )(inv, snd_part)
    return (M_img, M_snd)
